# baseline (speedup 1.0000x reference)
import functools

import jax
import jax.numpy as jnp
from jax import lax
from jax.experimental import pallas as pl
from jax.experimental.pallas import tpu as pltpu
from jax.experimental.pallas import tpu_sc as plsc

B = 16384
ED = 16
NU = 1000000
NC = 2
NS = 16
NW = NC * NS
BPW = B // NW
SLOT = 131072
CBR = 4096


def _tr_body(*refs):
    eye_ref, o_ref = refs[8], refs[9]
    x = jnp.concatenate([refs[s][...] for s in range(8)], axis=0)
    o_ref[...] = jax.lax.dot_general(
        x, eye_ref[...], (((0,), (0,)), ((), ())),
        preferred_element_type=jnp.float32,
        precision=jax.lax.Precision.HIGHEST)


def _detile(t):
    tt = t.T
    nsteps = SLOT // CBR
    last = (NU - 1) // CBR

    def spec(s):
        return pl.BlockSpec(
            (16, CBR),
            lambda r, s=s: (0, jnp.minimum(s * nsteps + r, last)))

    return pl.pallas_call(
        _tr_body,
        grid=(nsteps,),
        in_specs=[spec(s) for s in range(8)]
        + [pl.BlockSpec((128, 128), lambda r: (0, 0))],
        out_specs=pl.BlockSpec((CBR, 128), lambda r: (r, 0)),
        out_shape=jax.ShapeDtypeStruct((SLOT, 128), jnp.float32),
    )(*([tt] * 8 + [jnp.eye(128, dtype=jnp.float32)]))


def _sc_gather(uid, iid, age, gen, cat,
               user_t, item_t, age_t, gen_t, cat_t,
               out_u, out_i, out_a, out_g, out_c,
               uidx_v, iidx_v, aidx_v, gidx_v, cidx_v,
               urow_v, irow_v, arow_v, grow_v, crow_v,
               s0, s1, s2, s3, s4):
    wid = lax.axis_index("s") * NC + lax.axis_index("c")
    base = wid * BPW

    pltpu.sync_copy(uid.at[pl.ds(base, BPW)], uidx_v)
    pltpu.sync_copy(iid.at[pl.ds(base, BPW)], iidx_v)
    pltpu.sync_copy(age.at[pl.ds(base, BPW)], aidx_v)
    pltpu.sync_copy(gen.at[pl.ds(base, BPW)], gidx_v)
    pltpu.sync_copy(cat.at[pl.ds(base, BPW)], cidx_v)

    cu = pltpu.async_copy(user_t.at[uidx_v], urow_v, s0)
    ci = pltpu.async_copy(item_t.at[iidx_v], irow_v, s1)
    ca = pltpu.async_copy(age_t.at[aidx_v], arow_v, s2)
    cg = pltpu.async_copy(gen_t.at[gidx_v], grow_v, s3)
    cc = pltpu.async_copy(cat_t.at[cidx_v], crow_v, s4)
    cu.wait()
    ci.wait()
    ca.wait()
    cg.wait()
    cc.wait()

    pltpu.sync_copy(urow_v, out_u.at[pl.ds(base, BPW)])
    pltpu.sync_copy(irow_v, out_i.at[pl.ds(base, BPW)])
    pltpu.sync_copy(arow_v, out_a.at[pl.ds(base, BPW)])
    pltpu.sync_copy(grow_v, out_g.at[pl.ds(base, BPW)])
    pltpu.sync_copy(crow_v, out_c.at[pl.ds(base, BPW)])


_gather_call = functools.partial(
    pl.kernel,
    out_type=[jax.ShapeDtypeStruct((B, ED), jnp.float32)] * 5,
    mesh=plsc.VectorSubcoreMesh(core_axis_name="c", subcore_axis_name="s",
                                num_cores=NC, num_subcores=NS),
    scratch_types=(
        [pltpu.VMEM((BPW,), jnp.int32)] * 5
        + [pltpu.VMEM((BPW, ED), jnp.float32)] * 5
        + [pltpu.SemaphoreType.DMA] * 5
    ),
    compiler_params=pltpu.CompilerParams(use_tc_tiling_on_sc=False),
)(_sc_gather)


def _mlp_body(u_ref, i_ref, a_ref, g_ref, c_ref, p_ref,
              w1_ref, w1p_ref, b1_ref, w2_ref, b2_ref, w3_ref, b3_ref,
              o_ref):
    hp = jax.lax.Precision.HIGHEST
    u, it = u_ref[...], i_ref[...]
    a, g, c = a_ref[...], g_ref[...], c_ref[...]
    p = p_ref[...]
    w1, w1p, b1 = w1_ref[...], w1p_ref[...], b1_ref[...]
    w2, b2, w3, b3 = w2_ref[...], b2_ref[...], w3_ref[...], b3_ref[...]
    outs = []
    for k in range(8):
        s = slice(k * ED, (k + 1) * ED)
        x = jnp.concatenate([u[:, s], it[:, s], a[:, s], g[:, s], c[:, s]],
                            axis=1)
        h = jnp.dot(x, w1, preferred_element_type=jnp.float32, precision=hp)
        h = h + (p[:, k:k + 1] * 0.001) * w1p
        h = jnp.maximum(h + b1, 0.0)
        h = jnp.dot(h, w2, preferred_element_type=jnp.float32, precision=hp)
        h = jnp.maximum(h + b2, 0.0)
        outs.append(
            jnp.dot(h, w3, preferred_element_type=jnp.float32, precision=hp)
            + b3)
    o_ref[...] = jnp.concatenate(outs, axis=1)


def _mlp(u, i, a, g, c, p, w1s, w1p, b1, w2, b2, w3, b3, bt8=512):
    grid = (B // 8 // bt8,)
    feat_spec = pl.BlockSpec((bt8, 128), lambda j: (j, 0))
    full = lambda shape: pl.BlockSpec(shape, lambda j: (0, 0))
    return pl.pallas_call(
        _mlp_body,
        grid=grid,
        in_specs=[feat_spec] * 5 + [
            pl.BlockSpec((bt8, 8), lambda j: (j, 0)),
            full((5 * ED, 64)), full((1, 64)), full((1, 64)),
            full((64, 32)), full((1, 32)),
            full((32, 1)), full((1, 1)),
        ],
        out_specs=pl.BlockSpec((bt8, 8), lambda j: (j, 0)),
        out_shape=jax.ShapeDtypeStruct((B // 8, 8), jnp.float32),
    )(u, i, a, g, c, p, w1s, w1p, b1, w2, b2, w3, b3)


def kernel(user_id, age, gender, item_id, category, price,
           user_table, item_table, age_table, gender_table, cat_table,
           W1, b1, W2, b2, W3, b3):
    uid = user_id.reshape(B).astype(jnp.int32)
    iid = item_id.reshape(B).astype(jnp.int32)
    uid = ((uid & (SLOT - 1)) << 3) | (uid >> 17)
    iid = ((iid & (SLOT - 1)) << 3) | (iid >> 17)
    aid = age.reshape(B).astype(jnp.int32)
    gid = gender.reshape(B).astype(jnp.int32)
    cid = category.reshape(B).astype(jnp.int32)

    ut_lin = _detile(user_table).reshape(SLOT * 8, ED)
    it_lin = _detile(item_table).reshape(SLOT * 8, ED)

    age_p = jnp.pad(age_table, ((0, 0), (0, ED - age_table.shape[1])))
    gen_p = jnp.pad(gender_table, ((0, 0), (0, ED - gender_table.shape[1])))
    cat_p = jnp.pad(cat_table, ((0, 0), (0, ED - cat_table.shape[1])))

    u, i, a, g, c = _gather_call(
        uid, iid, aid, gid, cid,
        ut_lin, it_lin, age_p, gen_p, cat_p)

    z = jnp.zeros((1, 64), dtype=W1.dtype)
    w1s = jnp.concatenate([
        W1[0:32],
        W1[32:40], jnp.tile(z, (8, 1)),
        W1[40:44], jnp.tile(z, (12, 1)),
        W1[44:52], jnp.tile(z, (8, 1)),
    ], axis=0)

    out = _mlp(u.reshape(B // 8, 128), i.reshape(B // 8, 128),
               a.reshape(B // 8, 128), g.reshape(B // 8, 128),
               c.reshape(B // 8, 128), price.reshape(B // 8, 8),
               w1s, W1[52:53],
               b1.reshape(1, 64), W2, b2.reshape(1, 32),
               W3, b3.reshape(1, 1))
    return out.reshape(B)


# speedup vs baseline: 1.0581x; 1.0581x over previous
import functools

import jax
import jax.numpy as jnp
from jax import lax
from jax.experimental import pallas as pl
from jax.experimental.pallas import tpu as pltpu
from jax.experimental.pallas import tpu_sc as plsc

B = 16384
ED = 16
NU = 1000000
NC = 2
NS = 16
NW = NC * NS
BPW = B // NW
SLOT = 131072
CBR = 4096


def _tr_body(*refs):
    eye_ref, o_ref = refs[8], refs[9]
    x = jnp.concatenate([refs[s][...] for s in range(8)], axis=0)
    o_ref[...] = jax.lax.dot_general(
        x, eye_ref[...], (((0,), (0,)), ((), ())),
        preferred_element_type=jnp.float32,
        precision=jax.lax.Precision.HIGHEST)


def _detile(t):
    tt = t.T
    nsteps = SLOT // CBR
    last = (NU - 1) // CBR

    def spec(s):
        return pl.BlockSpec(
            (16, CBR),
            lambda r, s=s: (0, jnp.minimum(s * nsteps + r, last)))

    return pl.pallas_call(
        _tr_body,
        grid=(nsteps,),
        in_specs=[spec(s) for s in range(8)]
        + [pl.BlockSpec((128, 128), lambda r: (0, 0))],
        out_specs=pl.BlockSpec((CBR, 128), lambda r: (r, 0)),
        out_shape=jax.ShapeDtypeStruct((SLOT, 128), jnp.float32),
    )(*([tt] * 8 + [jnp.eye(128, dtype=jnp.float32)]))


def _sc_gather(uid, iid, age, gen, cat,
               user_t, item_t, age_t, gen_t, cat_t,
               out_u, out_i, out_a, out_g, out_c,
               uidx_v, iidx_v, aidx_v, gidx_v, cidx_v,
               urow_v, irow_v, arow_v, grow_v, crow_v,
               s0, s1, s2, s3, s4):
    wid = lax.axis_index("s") * NC + lax.axis_index("c")
    base = wid * BPW

    pltpu.sync_copy(uid.at[pl.ds(base, BPW)], uidx_v)
    pltpu.sync_copy(iid.at[pl.ds(base, BPW)], iidx_v)
    pltpu.sync_copy(age.at[pl.ds(base, BPW)], aidx_v)
    pltpu.sync_copy(gen.at[pl.ds(base, BPW)], gidx_v)
    pltpu.sync_copy(cat.at[pl.ds(base, BPW)], cidx_v)

    cu = pltpu.async_copy(user_t.at[uidx_v], urow_v, s0)
    ci = pltpu.async_copy(item_t.at[iidx_v], irow_v, s1)
    ca = pltpu.async_copy(age_t.at[aidx_v], arow_v, s2)
    cg = pltpu.async_copy(gen_t.at[gidx_v], grow_v, s3)
    cc = pltpu.async_copy(cat_t.at[cidx_v], crow_v, s4)
    cu.wait()
    ci.wait()
    ca.wait()
    cg.wait()
    cc.wait()

    pltpu.sync_copy(urow_v, out_u.at[pl.ds(base, BPW)])
    pltpu.sync_copy(irow_v, out_i.at[pl.ds(base, BPW)])
    pltpu.sync_copy(arow_v, out_a.at[pl.ds(base, BPW)])
    pltpu.sync_copy(grow_v, out_g.at[pl.ds(base, BPW)])
    pltpu.sync_copy(crow_v, out_c.at[pl.ds(base, BPW)])


_gather_call = functools.partial(
    pl.kernel,
    out_type=[jax.ShapeDtypeStruct((B, ED), jnp.float32)] * 5,
    mesh=plsc.VectorSubcoreMesh(core_axis_name="c", subcore_axis_name="s",
                                num_cores=NC, num_subcores=NS),
    scratch_types=(
        [pltpu.VMEM((BPW,), jnp.int32)] * 5
        + [pltpu.VMEM((BPW, ED), jnp.float32)] * 5
        + [pltpu.SemaphoreType.DMA] * 5
    ),
    compiler_params=pltpu.CompilerParams(use_tc_tiling_on_sc=False),
)(_sc_gather)


def _mlp_body(u_ref, i_ref, a_ref, g_ref, c_ref, p_ref,
              w1_ref, b1_ref, w2_ref, b2_ref, w3_ref, b3_ref,
              o_ref):
    u, it = u_ref[...], i_ref[...]
    a, g, c = a_ref[...], g_ref[...], c_ref[...]
    p = p_ref[...]
    w1, b1 = w1_ref[...], b1_ref[...]
    w2, b2, w3, b3 = w2_ref[...], b2_ref[...], w3_ref[...], b3_ref[...]
    zpad = jnp.zeros((u.shape[0], ED - 1), jnp.float32)
    outs = []
    for k in range(8):
        s = slice(k * ED, (k + 1) * ED)
        pg = jnp.concatenate([p[:, k:k + 1] / 1000.0, zpad], axis=1)
        x = jnp.concatenate(
            [u[:, s], it[:, s], a[:, s], g[:, s], c[:, s], pg], axis=1)
        h = jnp.dot(x, w1, preferred_element_type=jnp.float32)
        h = jnp.maximum(h + b1, 0.0)
        h = jnp.dot(h, w2, preferred_element_type=jnp.float32)
        h = jnp.maximum(h + b2, 0.0)
        outs.append(jnp.dot(h, w3, preferred_element_type=jnp.float32) + b3)
    o_ref[...] = jnp.concatenate(outs, axis=1)


def _mlp(u, i, a, g, c, p, w1s, b1, w2, b2, w3, b3, bt8=512):
    grid = (B // 8 // bt8,)
    feat_spec = pl.BlockSpec((bt8, 128), lambda j: (j, 0))
    full = lambda shape: pl.BlockSpec(shape, lambda j: (0, 0))
    return pl.pallas_call(
        _mlp_body,
        grid=grid,
        in_specs=[feat_spec] * 5 + [
            pl.BlockSpec((bt8, 8), lambda j: (j, 0)),
            full((6 * ED, 64)), full((1, 64)),
            full((64, 32)), full((1, 32)),
            full((32, 1)), full((1, 1)),
        ],
        out_specs=pl.BlockSpec((bt8, 8), lambda j: (j, 0)),
        out_shape=jax.ShapeDtypeStruct((B // 8, 8), jnp.float32),
    )(u, i, a, g, c, p, w1s, b1, w2, b2, w3, b3)


def kernel(user_id, age, gender, item_id, category, price,
           user_table, item_table, age_table, gender_table, cat_table,
           W1, b1, W2, b2, W3, b3):
    uid = user_id.reshape(B).astype(jnp.int32)
    iid = item_id.reshape(B).astype(jnp.int32)
    uid = ((uid & (SLOT - 1)) << 3) | (uid >> 17)
    iid = ((iid & (SLOT - 1)) << 3) | (iid >> 17)
    aid = age.reshape(B).astype(jnp.int32)
    gid = gender.reshape(B).astype(jnp.int32)
    cid = category.reshape(B).astype(jnp.int32)

    ut_lin = _detile(user_table).reshape(SLOT * 8, ED)
    it_lin = _detile(item_table).reshape(SLOT * 8, ED)

    age_p = jnp.pad(age_table, ((0, 0), (0, ED - age_table.shape[1])))
    gen_p = jnp.pad(gender_table, ((0, 0), (0, ED - gender_table.shape[1])))
    cat_p = jnp.pad(cat_table, ((0, 0), (0, ED - cat_table.shape[1])))

    u, i, a, g, c = _gather_call(
        uid, iid, aid, gid, cid,
        ut_lin, it_lin, age_p, gen_p, cat_p)

    z = jnp.zeros((1, 64), dtype=W1.dtype)
    w1s = jnp.concatenate([
        W1[0:32],
        W1[32:40], jnp.tile(z, (8, 1)),
        W1[40:44], jnp.tile(z, (12, 1)),
        W1[44:52], jnp.tile(z, (8, 1)),
        W1[52:53], jnp.tile(z, (15, 1)),
    ], axis=0)

    out = _mlp(u.reshape(B // 8, 128), i.reshape(B // 8, 128),
               a.reshape(B // 8, 128), g.reshape(B // 8, 128),
               c.reshape(B // 8, 128), price.reshape(B // 8, 8),
               w1s,
               b1.reshape(1, 64), W2, b2.reshape(1, 32),
               W3, b3.reshape(1, 1))
    return out.reshape(B)


# speedup vs baseline: 1.2437x; 1.1754x over previous
import functools

import jax
import jax.numpy as jnp
from jax import lax
from jax.experimental import pallas as pl
from jax.experimental.pallas import tpu as pltpu
from jax.experimental.pallas import tpu_sc as plsc

B = 16384
ED = 16
NU = 1000000
NC = 2
NS = 16
NW = NC * NS
BPW = B // NW
SLOT = 131072
CBR = 4096


def _tr_body(*refs):
    eye_ref, o_ref = refs[8], refs[9]
    x = jnp.concatenate([refs[s][...] for s in range(8)], axis=0)
    o_ref[...] = jax.lax.dot_general(
        x, eye_ref[...], (((0,), (0,)), ((), ())),
        preferred_element_type=jnp.float32)


def _detile(t):
    tt = t.T
    nsteps = SLOT // CBR
    last = (NU - 1) // CBR

    def spec(s):
        return pl.BlockSpec(
            (16, CBR),
            lambda r, s=s: (0, jnp.minimum(s * nsteps + r, last)))

    return pl.pallas_call(
        _tr_body,
        grid=(nsteps,),
        in_specs=[spec(s) for s in range(8)]
        + [pl.BlockSpec((128, 128), lambda r: (0, 0))],
        out_specs=pl.BlockSpec((CBR, 128), lambda r: (r, 0)),
        out_shape=jax.ShapeDtypeStruct((SLOT, 128), jnp.float32),
    )(*([tt] * 8 + [jnp.eye(128, dtype=jnp.float32)]))


def _sc_gather(uid, iid, age, gen, cat,
               user_t, item_t, age_t, gen_t, cat_t,
               out_u, out_i, out_a, out_g, out_c,
               uidx_v, iidx_v, aidx_v, gidx_v, cidx_v,
               urow_v, irow_v, arow_v, grow_v, crow_v,
               s0, s1, s2, s3, s4):
    wid = lax.axis_index("s") * NC + lax.axis_index("c")
    base = wid * BPW

    pltpu.sync_copy(uid.at[pl.ds(base, BPW)], uidx_v)
    pltpu.sync_copy(iid.at[pl.ds(base, BPW)], iidx_v)
    pltpu.sync_copy(age.at[pl.ds(base, BPW)], aidx_v)
    pltpu.sync_copy(gen.at[pl.ds(base, BPW)], gidx_v)
    pltpu.sync_copy(cat.at[pl.ds(base, BPW)], cidx_v)

    cu = pltpu.async_copy(user_t.at[uidx_v], urow_v, s0)
    ci = pltpu.async_copy(item_t.at[iidx_v], irow_v, s1)
    ca = pltpu.async_copy(age_t.at[aidx_v], arow_v, s2)
    cg = pltpu.async_copy(gen_t.at[gidx_v], grow_v, s3)
    cc = pltpu.async_copy(cat_t.at[cidx_v], crow_v, s4)
    cu.wait()
    ci.wait()
    ca.wait()
    cg.wait()
    cc.wait()

    pltpu.sync_copy(urow_v, out_u.at[pl.ds(base, BPW)])
    pltpu.sync_copy(irow_v, out_i.at[pl.ds(base, BPW)])
    pltpu.sync_copy(arow_v, out_a.at[pl.ds(base, BPW)])
    pltpu.sync_copy(grow_v, out_g.at[pl.ds(base, BPW)])
    pltpu.sync_copy(crow_v, out_c.at[pl.ds(base, BPW)])


_gather_call = functools.partial(
    pl.kernel,
    out_type=[jax.ShapeDtypeStruct((B, ED), jnp.float32)] * 5,
    mesh=plsc.VectorSubcoreMesh(core_axis_name="c", subcore_axis_name="s",
                                num_cores=NC, num_subcores=NS),
    scratch_types=(
        [pltpu.VMEM((BPW,), jnp.int32)] * 5
        + [pltpu.VMEM((BPW, ED), jnp.float32)] * 5
        + [pltpu.SemaphoreType.DMA] * 5
    ),
    compiler_params=pltpu.CompilerParams(use_tc_tiling_on_sc=False),
)(_sc_gather)


def _mlp_body(u_ref, i_ref, a_ref, g_ref, c_ref, p_ref,
              w1_ref, b1_ref, w2_ref, b2_ref, w3_ref, b3_ref,
              o_ref):
    u, it = u_ref[...], i_ref[...]
    a, g, c = a_ref[...], g_ref[...], c_ref[...]
    p = p_ref[...]
    w1, b1 = w1_ref[...], b1_ref[...]
    w2, b2, w3, b3 = w2_ref[...], b2_ref[...], w3_ref[...], b3_ref[...]
    zpad = jnp.zeros((u.shape[0], ED - 1), jnp.float32)
    outs = []
    for k in range(8):
        s = slice(k * ED, (k + 1) * ED)
        pg = jnp.concatenate([p[:, k:k + 1] / 1000.0, zpad], axis=1)
        x = jnp.concatenate(
            [u[:, s], it[:, s], a[:, s], g[:, s], c[:, s], pg], axis=1)
        h = jnp.dot(x, w1, preferred_element_type=jnp.float32)
        h = jnp.maximum(h + b1, 0.0)
        h = jnp.dot(h, w2, preferred_element_type=jnp.float32)
        h = jnp.maximum(h + b2, 0.0)
        outs.append(jnp.dot(h, w3, preferred_element_type=jnp.float32) + b3)
    o_ref[...] = jnp.concatenate(outs, axis=1)


def _mlp(u, i, a, g, c, p, w1s, b1, w2, b2, w3, b3, bt8=512):
    grid = (B // 8 // bt8,)
    feat_spec = pl.BlockSpec((bt8, 128), lambda j: (j, 0))
    full = lambda shape: pl.BlockSpec(shape, lambda j: (0, 0))
    return pl.pallas_call(
        _mlp_body,
        grid=grid,
        in_specs=[feat_spec] * 5 + [
            pl.BlockSpec((bt8, 8), lambda j: (j, 0)),
            full((6 * ED, 64)), full((1, 64)),
            full((64, 32)), full((1, 32)),
            full((32, 1)), full((1, 1)),
        ],
        out_specs=pl.BlockSpec((bt8, 8), lambda j: (j, 0)),
        out_shape=jax.ShapeDtypeStruct((B // 8, 8), jnp.float32),
    )(u, i, a, g, c, p, w1s, b1, w2, b2, w3, b3)


def kernel(user_id, age, gender, item_id, category, price,
           user_table, item_table, age_table, gender_table, cat_table,
           W1, b1, W2, b2, W3, b3):
    uid = user_id.reshape(B).astype(jnp.int32)
    iid = item_id.reshape(B).astype(jnp.int32)
    uid = ((uid & (SLOT - 1)) << 3) | (uid >> 17)
    iid = ((iid & (SLOT - 1)) << 3) | (iid >> 17)
    aid = age.reshape(B).astype(jnp.int32)
    gid = gender.reshape(B).astype(jnp.int32)
    cid = category.reshape(B).astype(jnp.int32)

    ut_lin = _detile(user_table).reshape(SLOT * 8, ED)
    it_lin = _detile(item_table).reshape(SLOT * 8, ED)

    age_p = jnp.pad(age_table, ((0, 0), (0, ED - age_table.shape[1])))
    gen_p = jnp.pad(gender_table, ((0, 0), (0, ED - gender_table.shape[1])))
    cat_p = jnp.pad(cat_table, ((0, 0), (0, ED - cat_table.shape[1])))

    u, i, a, g, c = _gather_call(
        uid, iid, aid, gid, cid,
        ut_lin, it_lin, age_p, gen_p, cat_p)

    z = jnp.zeros((1, 64), dtype=W1.dtype)
    w1s = jnp.concatenate([
        W1[0:32],
        W1[32:40], jnp.tile(z, (8, 1)),
        W1[40:44], jnp.tile(z, (12, 1)),
        W1[44:52], jnp.tile(z, (8, 1)),
        W1[52:53], jnp.tile(z, (15, 1)),
    ], axis=0)

    out = _mlp(u.reshape(B // 8, 128), i.reshape(B // 8, 128),
               a.reshape(B // 8, 128), g.reshape(B // 8, 128),
               c.reshape(B // 8, 128), price.reshape(B // 8, 8),
               w1s,
               b1.reshape(1, 64), W2, b2.reshape(1, 32),
               W3, b3.reshape(1, 1))
    return out.reshape(B)


# speedup vs baseline: 1.2520x; 1.0066x over previous
import functools

import jax
import jax.numpy as jnp
from jax import lax
from jax.experimental import pallas as pl
from jax.experimental.pallas import tpu as pltpu
from jax.experimental.pallas import tpu_sc as plsc

B = 16384
ED = 16
NU = 1000000
NC = 2
NS = 16
NW = NC * NS
BPW = B // NW
SLOT = 131072
CBR = 4096


def _tr_body(*refs):
    eye_ref, o_ref = refs[8], refs[9]
    x = jnp.concatenate([refs[s][...] for s in range(8)], axis=0)
    o_ref[...] = jax.lax.dot_general(
        x, eye_ref[...], (((0,), (0,)), ((), ())),
        preferred_element_type=jnp.float32)


def _detile(t):
    tt = t.T
    nsteps = SLOT // CBR
    last = (NU - 1) // CBR

    def spec(s):
        return pl.BlockSpec(
            (16, CBR),
            lambda r, s=s: (0, jnp.minimum(s * nsteps + r, last)))

    return pl.pallas_call(
        _tr_body,
        grid=(nsteps,),
        in_specs=[spec(s) for s in range(8)]
        + [pl.BlockSpec((128, 128), lambda r: (0, 0))],
        out_specs=pl.BlockSpec((CBR, 128), lambda r: (r, 0)),
        out_shape=jax.ShapeDtypeStruct((SLOT, 128), jnp.float32),
    )(*([tt] * 8 + [jnp.eye(128, dtype=jnp.float32)]))


def _sc_gather_u(uid, user_t, out_u, uidx_v, urow_v, s0):
    wid = lax.axis_index("s") * NC + lax.axis_index("c")
    base = wid * BPW
    pltpu.sync_copy(uid.at[pl.ds(base, BPW)], uidx_v)
    pltpu.async_copy(user_t.at[uidx_v], urow_v, s0).wait()
    pltpu.sync_copy(urow_v, out_u.at[pl.ds(base, BPW)])


def _sc_gather_rest(iid, age, gen, cat,
                    item_t, age_t, gen_t, cat_t,
                    out_i, out_a, out_g, out_c,
                    iidx_v, aidx_v, gidx_v, cidx_v,
                    irow_v, arow_v, grow_v, crow_v,
                    s1, s2, s3, s4):
    wid = lax.axis_index("s") * NC + lax.axis_index("c")
    base = wid * BPW

    pltpu.sync_copy(iid.at[pl.ds(base, BPW)], iidx_v)
    pltpu.sync_copy(age.at[pl.ds(base, BPW)], aidx_v)
    pltpu.sync_copy(gen.at[pl.ds(base, BPW)], gidx_v)
    pltpu.sync_copy(cat.at[pl.ds(base, BPW)], cidx_v)

    ci = pltpu.async_copy(item_t.at[iidx_v], irow_v, s1)
    ca = pltpu.async_copy(age_t.at[aidx_v], arow_v, s2)
    cg = pltpu.async_copy(gen_t.at[gidx_v], grow_v, s3)
    cc = pltpu.async_copy(cat_t.at[cidx_v], crow_v, s4)
    ci.wait()
    ca.wait()
    cg.wait()
    cc.wait()

    pltpu.sync_copy(irow_v, out_i.at[pl.ds(base, BPW)])
    pltpu.sync_copy(arow_v, out_a.at[pl.ds(base, BPW)])
    pltpu.sync_copy(grow_v, out_g.at[pl.ds(base, BPW)])
    pltpu.sync_copy(crow_v, out_c.at[pl.ds(base, BPW)])


_SC_MESH = plsc.VectorSubcoreMesh(core_axis_name="c", subcore_axis_name="s",
                                  num_cores=NC, num_subcores=NS)

_gather_u_call = functools.partial(
    pl.kernel,
    out_type=jax.ShapeDtypeStruct((B, ED), jnp.float32),
    mesh=_SC_MESH,
    scratch_types=(
        [pltpu.VMEM((BPW,), jnp.int32),
         pltpu.VMEM((BPW, ED), jnp.float32),
         pltpu.SemaphoreType.DMA]
    ),
    compiler_params=pltpu.CompilerParams(use_tc_tiling_on_sc=False),
)(_sc_gather_u)

_gather_rest_call = functools.partial(
    pl.kernel,
    out_type=[jax.ShapeDtypeStruct((B, ED), jnp.float32)] * 4,
    mesh=_SC_MESH,
    scratch_types=(
        [pltpu.VMEM((BPW,), jnp.int32)] * 4
        + [pltpu.VMEM((BPW, ED), jnp.float32)] * 4
        + [pltpu.SemaphoreType.DMA] * 4
    ),
    compiler_params=pltpu.CompilerParams(use_tc_tiling_on_sc=False),
)(_sc_gather_rest)


def _mlp_body(u_ref, i_ref, a_ref, g_ref, c_ref, p_ref,
              w1_ref, b1_ref, w2_ref, b2_ref, w3_ref, b3_ref,
              o_ref):
    u, it = u_ref[...], i_ref[...]
    a, g, c = a_ref[...], g_ref[...], c_ref[...]
    p = p_ref[...]
    w1, b1 = w1_ref[...], b1_ref[...]
    w2, b2, w3, b3 = w2_ref[...], b2_ref[...], w3_ref[...], b3_ref[...]
    zpad = jnp.zeros((u.shape[0], ED - 1), jnp.float32)
    outs = []
    for k in range(8):
        s = slice(k * ED, (k + 1) * ED)
        pg = jnp.concatenate([p[:, k:k + 1] / 1000.0, zpad], axis=1)
        x = jnp.concatenate(
            [u[:, s], it[:, s], a[:, s], g[:, s], c[:, s], pg], axis=1)
        h = jnp.dot(x, w1, preferred_element_type=jnp.float32)
        h = jnp.maximum(h + b1, 0.0)
        h = jnp.dot(h, w2, preferred_element_type=jnp.float32)
        h = jnp.maximum(h + b2, 0.0)
        outs.append(jnp.dot(h, w3, preferred_element_type=jnp.float32) + b3)
    o_ref[...] = jnp.concatenate(outs, axis=1)


def _mlp(u, i, a, g, c, p, w1s, b1, w2, b2, w3, b3, bt8=512):
    grid = (B // 8 // bt8,)
    feat_spec = pl.BlockSpec((bt8, 128), lambda j: (j, 0))
    full = lambda shape: pl.BlockSpec(shape, lambda j: (0, 0))
    return pl.pallas_call(
        _mlp_body,
        grid=grid,
        in_specs=[feat_spec] * 5 + [
            pl.BlockSpec((bt8, 8), lambda j: (j, 0)),
            full((6 * ED, 64)), full((1, 64)),
            full((64, 32)), full((1, 32)),
            full((32, 1)), full((1, 1)),
        ],
        out_specs=pl.BlockSpec((bt8, 8), lambda j: (j, 0)),
        out_shape=jax.ShapeDtypeStruct((B // 8, 8), jnp.float32),
    )(u, i, a, g, c, p, w1s, b1, w2, b2, w3, b3)


def kernel(user_id, age, gender, item_id, category, price,
           user_table, item_table, age_table, gender_table, cat_table,
           W1, b1, W2, b2, W3, b3):
    uid = user_id.reshape(B).astype(jnp.int32)
    iid = item_id.reshape(B).astype(jnp.int32)
    uid = ((uid & (SLOT - 1)) << 3) | (uid >> 17)
    iid = ((iid & (SLOT - 1)) << 3) | (iid >> 17)
    aid = age.reshape(B).astype(jnp.int32)
    gid = gender.reshape(B).astype(jnp.int32)
    cid = category.reshape(B).astype(jnp.int32)

    ut_lin = _detile(user_table).reshape(SLOT * 8, ED)
    u = _gather_u_call(uid, ut_lin)
    it_lin = _detile(item_table).reshape(SLOT * 8, ED)

    age_p = jnp.pad(age_table, ((0, 0), (0, ED - age_table.shape[1])))
    gen_p = jnp.pad(gender_table, ((0, 0), (0, ED - gender_table.shape[1])))
    cat_p = jnp.pad(cat_table, ((0, 0), (0, ED - cat_table.shape[1])))

    i, a, g, c = _gather_rest_call(
        iid, aid, gid, cid, it_lin, age_p, gen_p, cat_p)

    z = jnp.zeros((1, 64), dtype=W1.dtype)
    w1s = jnp.concatenate([
        W1[0:32],
        W1[32:40], jnp.tile(z, (8, 1)),
        W1[40:44], jnp.tile(z, (12, 1)),
        W1[44:52], jnp.tile(z, (8, 1)),
        W1[52:53], jnp.tile(z, (15, 1)),
    ], axis=0)

    out = _mlp(u.reshape(B // 8, 128), i.reshape(B // 8, 128),
               a.reshape(B // 8, 128), g.reshape(B // 8, 128),
               c.reshape(B // 8, 128), price.reshape(B // 8, 8),
               w1s,
               b1.reshape(1, 64), W2, b2.reshape(1, 32),
               W3, b3.reshape(1, 1))
    return out.reshape(B)


# speedup vs baseline: 1.5939x; 1.2731x over previous
import functools

import jax
import jax.numpy as jnp
from jax import lax
from jax.experimental import pallas as pl
from jax.experimental.pallas import tpu as pltpu
from jax.experimental.pallas import tpu_sc as plsc

B = 16384
ED = 16
NU = 1000000
NC = 2
NS = 16
NW = NC * NS
BPW = B // NW
SLOT = 131072
CBR = 4096


def _tr_body(*refs):
    eye_ref, o_ref = refs[8], refs[9]
    x = jnp.concatenate([refs[s][...] for s in range(8)], axis=0)
    o_ref[...] = jax.lax.dot_general(
        x, eye_ref[...], (((0,), (0,)), ((), ())),
        preferred_element_type=jnp.float32)


def _detile(t):
    tt = t.T
    nsteps = SLOT // CBR
    last = (NU - 1) // CBR

    def spec(s):
        return pl.BlockSpec(
            (16, CBR),
            lambda r, s=s: (0, jnp.minimum(s * nsteps + r, last)))

    return pl.pallas_call(
        _tr_body,
        grid=(nsteps,),
        in_specs=[spec(s) for s in range(8)]
        + [pl.BlockSpec((128, 128), lambda r: (0, 0))],
        out_specs=pl.BlockSpec((CBR, 128), lambda r: (r, 0)),
        out_shape=jax.ShapeDtypeStruct((SLOT, 128), jnp.float32),
    )(*([tt] * 8 + [jnp.eye(128, dtype=jnp.float32)]))


def _sc_gather_u(uid, user_t, out_u, uidx_v, urow_v, s0):
    wid = lax.axis_index("s") * NC + lax.axis_index("c")
    base = wid * BPW
    pltpu.sync_copy(uid.at[pl.ds(base, BPW)], uidx_v)
    pltpu.async_copy(user_t.at[uidx_v], urow_v, s0).wait()
    pltpu.sync_copy(urow_v, out_u.at[pl.ds(base, BPW)])


def _sc_gather_small(age, gen, cat,
                     age_t, gen_t, cat_t,
                     out_a, out_g, out_c,
                     aidx_v, gidx_v, cidx_v,
                     arow_v, grow_v, crow_v,
                     s2, s3, s4):
    wid = lax.axis_index("s") * NC + lax.axis_index("c")
    base = wid * BPW

    pltpu.sync_copy(age.at[pl.ds(base, BPW)], aidx_v)
    pltpu.sync_copy(gen.at[pl.ds(base, BPW)], gidx_v)
    pltpu.sync_copy(cat.at[pl.ds(base, BPW)], cidx_v)

    ca = pltpu.async_copy(age_t.at[aidx_v], arow_v, s2)
    cg = pltpu.async_copy(gen_t.at[gidx_v], grow_v, s3)
    cc = pltpu.async_copy(cat_t.at[cidx_v], crow_v, s4)
    ca.wait()
    cg.wait()
    cc.wait()

    pltpu.sync_copy(arow_v, out_a.at[pl.ds(base, BPW)])
    pltpu.sync_copy(grow_v, out_g.at[pl.ds(base, BPW)])
    pltpu.sync_copy(crow_v, out_c.at[pl.ds(base, BPW)])


_SC_MESH = plsc.VectorSubcoreMesh(core_axis_name="c", subcore_axis_name="s",
                                  num_cores=NC, num_subcores=NS)

_gather_u_call = functools.partial(
    pl.kernel,
    out_type=jax.ShapeDtypeStruct((B, ED), jnp.float32),
    mesh=_SC_MESH,
    scratch_types=(
        [pltpu.VMEM((BPW,), jnp.int32),
         pltpu.VMEM((BPW, ED), jnp.float32),
         pltpu.SemaphoreType.DMA]
    ),
    compiler_params=pltpu.CompilerParams(use_tc_tiling_on_sc=False),
)(_sc_gather_u)

_gather_small_call = functools.partial(
    pl.kernel,
    out_type=[jax.ShapeDtypeStruct((B, ED), jnp.float32)] * 3,
    mesh=_SC_MESH,
    scratch_types=(
        [pltpu.VMEM((BPW,), jnp.int32)] * 3
        + [pltpu.VMEM((BPW, ED), jnp.float32)] * 3
        + [pltpu.SemaphoreType.DMA] * 3
    ),
    compiler_params=pltpu.CompilerParams(use_tc_tiling_on_sc=False),
)(_sc_gather_small)


def _mlp_body(u_ref, i_ref, a_ref, g_ref, c_ref, p_ref,
              w1_ref, b1_ref, w2_ref, b2_ref, w3_ref, b3_ref,
              o_ref):
    u, it = u_ref[...], i_ref[...]
    a, g, c = a_ref[...], g_ref[...], c_ref[...]
    p = p_ref[...]
    w1, b1 = w1_ref[...], b1_ref[...]
    w2, b2, w3, b3 = w2_ref[...], b2_ref[...], w3_ref[...], b3_ref[...]
    zpad = jnp.zeros((u.shape[0], ED - 1), jnp.float32)
    outs = []
    for k in range(8):
        s = slice(k * ED, (k + 1) * ED)
        pg = jnp.concatenate([p[:, k:k + 1] / 1000.0, zpad], axis=1)
        x = jnp.concatenate(
            [u[:, s], it[:, s], a[:, s], g[:, s], c[:, s], pg], axis=1)
        h = jnp.dot(x, w1, preferred_element_type=jnp.float32)
        h = jnp.maximum(h + b1, 0.0)
        h = jnp.dot(h, w2, preferred_element_type=jnp.float32)
        h = jnp.maximum(h + b2, 0.0)
        outs.append(jnp.dot(h, w3, preferred_element_type=jnp.float32) + b3)
    o_ref[...] = jnp.concatenate(outs, axis=1)


def _mlp(u, i, a, g, c, p, w1s, b1, w2, b2, w3, b3, bt8=512):
    grid = (B // 8 // bt8,)
    feat_spec = pl.BlockSpec((bt8, 128), lambda j: (j, 0))
    full = lambda shape: pl.BlockSpec(shape, lambda j: (0, 0))
    return pl.pallas_call(
        _mlp_body,
        grid=grid,
        in_specs=[feat_spec] * 5 + [
            pl.BlockSpec((bt8, 8), lambda j: (j, 0)),
            full((6 * ED, 64)), full((1, 64)),
            full((64, 32)), full((1, 32)),
            full((32, 1)), full((1, 1)),
        ],
        out_specs=pl.BlockSpec((bt8, 8), lambda j: (j, 0)),
        out_shape=jax.ShapeDtypeStruct((B // 8, 8), jnp.float32),
    )(u, i, a, g, c, p, w1s, b1, w2, b2, w3, b3)


def kernel(user_id, age, gender, item_id, category, price,
           user_table, item_table, age_table, gender_table, cat_table,
           W1, b1, W2, b2, W3, b3):
    uid = user_id.reshape(B).astype(jnp.int32)
    iid = item_id.reshape(B).astype(jnp.int32)
    uid = ((uid & (SLOT - 1)) << 3) | (uid >> 17)
    iid = ((iid & (SLOT - 1)) << 3) | (iid >> 17)
    aid = age.reshape(B).astype(jnp.int32)
    gid = gender.reshape(B).astype(jnp.int32)
    cid = category.reshape(B).astype(jnp.int32)

    rep = jnp.arange(B, dtype=jnp.int32)
    age_r = jnp.tile(jnp.pad(age_table, ((0, 0), (0, ED - 8))), (32, 1))
    gen_r = jnp.tile(jnp.pad(gender_table, ((0, 0), (0, ED - 4))), (512, 1))
    cat_r = jnp.tile(jnp.pad(cat_table, ((0, 0), (0, ED - 8))), (32, 1))
    aid = aid + (rep & 31) * 100
    gid = gid + (rep & 511) * 3
    cid = cid + (rep & 31) * 1000

    a, g, c = _gather_small_call(aid, gid, cid, age_r, gen_r, cat_r)
    ut_lin = _detile(user_table).reshape(SLOT * 8, ED)
    u = _gather_u_call(uid, ut_lin)
    it_lin = _detile(item_table).reshape(SLOT * 8, ED)
    i = _gather_u_call(iid, it_lin)

    z = jnp.zeros((1, 64), dtype=W1.dtype)
    w1s = jnp.concatenate([
        W1[0:32],
        W1[32:40], jnp.tile(z, (8, 1)),
        W1[40:44], jnp.tile(z, (12, 1)),
        W1[44:52], jnp.tile(z, (8, 1)),
        W1[52:53], jnp.tile(z, (15, 1)),
    ], axis=0)

    out = _mlp(u.reshape(B // 8, 128), i.reshape(B // 8, 128),
               a.reshape(B // 8, 128), g.reshape(B // 8, 128),
               c.reshape(B // 8, 128), price.reshape(B // 8, 8),
               w1s,
               b1.reshape(1, 64), W2, b2.reshape(1, 32),
               W3, b3.reshape(1, 1))
    return out.reshape(B)
